# baseline (speedup 1.0000x reference)
import functools

import jax
import jax.numpy as jnp
from jax import lax
from jax.experimental import pallas as pl
from jax.experimental.pallas import tpu as pltpu
from jax.experimental.pallas import tpu_sc as plsc

BATCH = 16384
FACTORS = 64
NC = 2
NS = 16
NW = NC * NS
PER_W = BATCH // NW
CHUNK = 128
NCHUNK = PER_W // CHUNK
GROUPS = CHUNK // 16


def _bpr_body(user_hbm, item_i_hbm, item_j_hbm, ue_hbm, ie_hbm, ib_hbm,
              out_hbm,
              idx_u, idx_i, idx_j, rows_u, rows_i, rows_j,
              bias_i, bias_j, tr, out_v, sem):
    wid = lax.axis_index("s") * NC + lax.axis_index("c")
    base = wid * PER_W
    iota = lax.iota(jnp.int32, 16)
    zeros16 = jnp.zeros((16,), jnp.int32)

    for c in range(NCHUNK):
        off = base + c * CHUNK
        pltpu.sync_copy(user_hbm.at[pl.ds(off, CHUNK)], idx_u.at[c])
        pltpu.sync_copy(item_i_hbm.at[pl.ds(off, CHUNK)], idx_i.at[c])
        pltpu.sync_copy(item_j_hbm.at[pl.ds(off, CHUNK)], idx_j.at[c])

    for c in range(NCHUNK):
        d1 = pltpu.async_copy(ue_hbm.at[idx_u.at[c]], rows_u, sem)
        d2 = pltpu.async_copy(ie_hbm.at[idx_i.at[c]], rows_i, sem)
        d3 = pltpu.async_copy(ie_hbm.at[idx_j.at[c]], rows_j, sem)
        d4 = pltpu.async_copy(ib_hbm.at[idx_i.at[c]], bias_i, sem)
        d5 = pltpu.async_copy(ib_hbm.at[idx_j.at[c]], bias_j, sem)
        d1.wait()
        d2.wait()
        d3.wait()
        d4.wait()
        d5.wait()

        def group_body(g, carry):
            for b16 in range(16):
                b = g * 16 + b16
                acc = None
                for k in range(4):
                    sl = pl.ds(k * 16, 16)
                    p = rows_u[b, sl] * (rows_i[b, sl] - rows_j[b, sl])
                    acc = p if acc is None else acc + p
                tr[pl.ds(b16 * 16, 16)] = acc
            iota16 = iota * 16
            tot = plsc.load_gather(tr, [iota16])
            for cc in range(1, 16):
                col = plsc.load_gather(tr, [iota16 + cc])
                tot = tot + col
            bi = bias_i[pl.ds(g * 16, 16)]
            bj = bias_j[pl.ds(g * 16, 16)]
            out_v[pl.ds(c * CHUNK + g * 16, 16)] = tot + bi - bj
            return carry

        lax.fori_loop(0, GROUPS, group_body, 0)

    pltpu.sync_copy(out_v, out_hbm.at[pl.ds(base, PER_W)])


_bpr_sc = functools.partial(
    pl.kernel,
    out_type=jax.ShapeDtypeStruct((BATCH,), jnp.float32),
    mesh=plsc.VectorSubcoreMesh(core_axis_name="c", subcore_axis_name="s"),
    compiler_params=pltpu.CompilerParams(needs_layout_passes=False,
                                         use_tc_tiling_on_sc=False),
    scratch_types=[
        pltpu.VMEM((NCHUNK, CHUNK), jnp.int32),
        pltpu.VMEM((NCHUNK, CHUNK), jnp.int32),
        pltpu.VMEM((NCHUNK, CHUNK), jnp.int32),
        pltpu.VMEM((CHUNK, FACTORS), jnp.float32),
        pltpu.VMEM((CHUNK, FACTORS), jnp.float32),
        pltpu.VMEM((CHUNK, FACTORS), jnp.float32),
        pltpu.VMEM((CHUNK,), jnp.float32),
        pltpu.VMEM((CHUNK,), jnp.float32),
        pltpu.VMEM((256,), jnp.float32),
        pltpu.VMEM((PER_W,), jnp.float32),
        pltpu.SemaphoreType.DMA,
    ],
)(_bpr_body)


def kernel(user, item_i, item_j, user_embedding, item_embedding, item_bias):
    return _bpr_sc(user.astype(jnp.int32), item_i.astype(jnp.int32),
                   item_j.astype(jnp.int32), user_embedding, item_embedding,
                   item_bias.reshape(-1))
